# Initial kernel scaffold; baseline (speedup 1.0000x reference)
import jax
import jax.numpy as jnp
from jax.experimental import pallas as pl


def kernel(x, mask, W):
    raise NotImplementedError("write your pallas kernel here")


# speedup vs baseline: 1.0252x; 1.0252x over previous
import functools

import jax
import jax.numpy as jnp
from jax import lax
from jax.experimental import pallas as pl
from jax.experimental.pallas import tpu as pltpu
from jax.experimental.pallas import tpu_sc as plsc

N = 32768
B = 128
K = 8
L = 16
NEG = float("-inf")

_info = plsc.get_sparse_core_info()
_NC, _NS = _info.num_cores, _info.num_subcores
_NW = _NC * _NS
_RPW = B // _NW

BLK = 128
VPB = BLK // L
NBLK = N // BLK


_GATHER_DNUMS = lax.GatherDimensionNumbers(
    offset_dims=(), collapsed_slice_dims=(0,), start_index_map=(0,)
)


def _shuf(v, idx):
    return lax.gather(
        v,
        idx[:, None],
        dimension_numbers=_GATHER_DNUMS,
        slice_sizes=(1,),
        mode=lax.GatherScatterMode.PROMISE_IN_BOUNDS,
    )


def _ball(v, op, iota):
    for sh in (1, 2, 4, 8):
        v = op(v, _shuf(v, iota ^ sh))
    return v


def _row_topk_combine(xv, mv, w_hbm, zinv, rsv, isv, idxv, wg, sem, iota):
    for s in range(K):
        rsv[pl.ds(s * L, L)] = jnp.full((L,), NEG, jnp.float32)
        isv[pl.ds(s * L, L)] = jnp.zeros((L,), jnp.int32)

    def blk_body(blk, carry):
        base = blk * BLK
        mx = jnp.full((L,), NEG)
        for j in range(VPB):
            v = xv[pl.ds(base + j * L, L)]
            m = mv[pl.ds(base + j * L, L)]
            mx = jnp.maximum(mx, jnp.where(m > 0, v, NEG))
        thr = rsv[pl.ds((K - 1) * L, L)]
        anyv = _ball(
            jnp.where(mx > thr, 1, 0).astype(jnp.int32), jnp.maximum, iota
        )

        @pl.when(anyv[0] > 0)
        def _():
            rs = [rsv[pl.ds(s * L, L)] for s in range(K)]
            idxs = [isv[pl.ds(s * L, L)] for s in range(K)]
            for j in range(VPB):
                v = xv[pl.ds(base + j * L, L)]
                m = mv[pl.ds(base + j * L, L)]
                cv = jnp.where(m > 0, v, NEG)
                ci = base + j * L + iota
                for s in range(K):
                    take = cv > rs[s]
                    nr = jnp.where(take, cv, rs[s])
                    ni = jnp.where(take, ci, idxs[s])
                    cv, ci = (
                        jnp.where(take, rs[s], cv),
                        jnp.where(take, idxs[s], ci),
                    )
                    rs[s], idxs[s] = nr, ni
            for s in range(K):
                rsv[pl.ds(s * L, L)] = rs[s]
                isv[pl.ds(s * L, L)] = idxs[s]

        return carry

    lax.fori_loop(0, NBLK, blk_body, 0)
    rs = [rsv[pl.ds(s * L, L)] for s in range(K)]
    idxs = [isv[pl.ds(s * L, L)] for s in range(K)]

    idxvec = jnp.zeros((L,), jnp.int32)
    valvec = jnp.zeros((L,), jnp.float32)
    for k in range(K):
        gmax = _ball(rs[0], jnp.maximum, iota)
        hit = rs[0] == gmax
        ml = _ball(jnp.where(hit, iota, L), jnp.minimum, iota)
        hm = iota == ml
        ik = _ball(jnp.where(hm, idxs[0], 0), jnp.maximum, iota)
        valvec = jnp.where(iota == k, gmax, valvec)
        idxvec = jnp.where(iota == k, ik, idxvec)
        for s in range(K - 1):
            rs[s] = jnp.where(hm, rs[s + 1], rs[s])
            idxs[s] = jnp.where(hm, idxs[s + 1], idxs[s])
        rs[K - 1] = jnp.where(hm, NEG, rs[K - 1])

    idxv[...] = idxvec
    pltpu.async_copy(w_hbm.at[idxv], wg, sem).wait()
    ew = jnp.exp(wg[...])
    lanemask = iota < K
    num = _ball(jnp.where(lanemask, valvec * ew, 0.0), jnp.add, iota)
    vsum = _ball(jnp.where(lanemask, valvec, 0.0), jnp.add, iota)
    return num * zinv + vsum / float(K)


@functools.partial(
    pl.kernel,
    mesh=plsc.VectorSubcoreMesh(core_axis_name="c", subcore_axis_name="s"),
    out_type=jax.ShapeDtypeStruct((_NW, L), jnp.float32),
    scratch_types=[
        pltpu.VMEM((N,), jnp.float32),
        pltpu.VMEM((N,), jnp.int32),
        pltpu.VMEM((N,), jnp.float32),
        pltpu.VMEM((L,), jnp.float32),
        pltpu.VMEM((K * L,), jnp.float32),
        pltpu.VMEM((K * L,), jnp.int32),
        pltpu.VMEM((L,), jnp.int32),
        pltpu.VMEM((L,), jnp.float32),
        pltpu.SemaphoreType.DMA,
    ],
)
def _sc_kernel(
    x_hbm, m_hbm, w_hbm, out_hbm, xv, mv, wv, resv, rsv, isv, idxv, wg, sem
):
    wid = lax.axis_index("s") * _NC + lax.axis_index("c")
    iota = lax.iota(jnp.int32, L)

    pltpu.sync_copy(w_hbm, wv)

    def zstep(i, acc):
        return acc + jnp.exp(wv[pl.ds(i * L, L)])

    zvec = lax.fori_loop(0, N // L, zstep, jnp.zeros((L,), jnp.float32))
    zinv = 1.0 / _ball(zvec, jnp.add, iota)

    resvec = jnp.zeros((L,), jnp.float32)
    for r_local in range(_RPW):
        row = wid * _RPW + r_local
        pltpu.sync_copy(x_hbm.at[pl.ds(row * N, N)], xv)
        pltpu.sync_copy(m_hbm.at[pl.ds(row * N, N)], mv)
        outb = _row_topk_combine(
            xv, mv, w_hbm, zinv, rsv, isv, idxv, wg, sem, iota
        )
        resvec = jnp.where(iota == r_local, outb, resvec)

    resv[...] = resvec
    pltpu.sync_copy(resv, out_hbm.at[wid])


def kernel(x, mask, W):
    res = _sc_kernel(
        x.reshape(-1),
        mask.astype(jnp.int32).reshape(-1),
        W.reshape(-1),
    )
    return res[:, :_RPW].reshape(B, 1)


# speedup vs baseline: 1.0607x; 1.0346x over previous
import functools

import jax
import jax.numpy as jnp
from jax import lax
from jax.experimental import pallas as pl
from jax.experimental.pallas import tpu as pltpu
from jax.experimental.pallas import tpu_sc as plsc

N = 32768
B = 128
K = 8
L = 16
POW = N // 4
NEG = float("-inf")

_info = plsc.get_sparse_core_info()
_NC, _NS = _info.num_cores, _info.num_subcores
_NW = _NC * _NS
_RPW = B // _NW

BLK = 128
NBLK = N // BLK

_GATHER_DNUMS = lax.GatherDimensionNumbers(
    offset_dims=(), collapsed_slice_dims=(0,), start_index_map=(0,)
)


def _shuf(v, idx):
    return lax.gather(
        v,
        idx[:, None],
        dimension_numbers=_GATHER_DNUMS,
        slice_sizes=(1,),
        mode=lax.GatherScatterMode.PROMISE_IN_BOUNDS,
    )


def _ball(v, op, iota):
    for sh in (1, 2, 4, 8):
        v = op(v, _shuf(v, iota ^ sh))
    return v


def _row_topk_combine(
    xv, mv, xo, mo, w_hbm, zinv, rsv, isv, idxv, wg, semw, iota, perms, shiftv
):
    for s in range(K):
        rsv[pl.ds(s * L, L)] = jnp.full((L,), NEG, jnp.float32)
        isv[pl.ds(s * L, L)] = jnp.zeros((L,), jnp.int32)

    def blk_body(blk, carry):
        base = blk * BLK
        xb = xo + base
        mb = mo + blk * (BLK // 4)
        mx = jnp.full((L,), NEG)
        for h in range(2):
            mw = mv[pl.ds(mb + h * L, L)]
            for j in range(4):
                v = xv[pl.ds(xb + h * 64 + j * L, L)]
                bits = (_shuf(mw, perms[j]) >> shiftv) & 1
                mx = jnp.maximum(mx, jnp.where(bits > 0, v, NEG))
        thr = rsv[pl.ds((K - 1) * L, L)]
        anyv = _ball(
            jnp.where(mx > thr, 1, 0).astype(jnp.int32), jnp.maximum, iota
        )

        @pl.when(anyv[0] > 0)
        def _():
            rs = [rsv[pl.ds(s * L, L)] for s in range(K)]
            idxs = [isv[pl.ds(s * L, L)] for s in range(K)]
            for h in range(2):
                mw = mv[pl.ds(mb + h * L, L)]
                for j in range(4):
                    v = xv[pl.ds(xb + h * 64 + j * L, L)]
                    bits = (_shuf(mw, perms[j]) >> shiftv) & 1
                    cv = jnp.where(bits > 0, v, NEG)
                    ci = base + h * 64 + j * L + iota
                    for s in range(K):
                        take = cv > rs[s]
                        nr = jnp.where(take, cv, rs[s])
                        ni = jnp.where(take, ci, idxs[s])
                        cv, ci = (
                            jnp.where(take, rs[s], cv),
                            jnp.where(take, idxs[s], ci),
                        )
                        rs[s], idxs[s] = nr, ni
            for s in range(K):
                rsv[pl.ds(s * L, L)] = rs[s]
                isv[pl.ds(s * L, L)] = idxs[s]

        return carry

    lax.fori_loop(0, NBLK, blk_body, 0)
    rs = [rsv[pl.ds(s * L, L)] for s in range(K)]
    idxs = [isv[pl.ds(s * L, L)] for s in range(K)]

    idxvec = jnp.zeros((L,), jnp.int32)
    valvec = jnp.zeros((L,), jnp.float32)
    for k in range(K):
        gmax = _ball(rs[0], jnp.maximum, iota)
        hit = rs[0] == gmax
        ml = _ball(jnp.where(hit, iota, L), jnp.minimum, iota)
        hm = iota == ml
        ik = _ball(jnp.where(hm, idxs[0], 0), jnp.maximum, iota)
        valvec = jnp.where(iota == k, gmax, valvec)
        idxvec = jnp.where(iota == k, ik, idxvec)
        for s in range(K - 1):
            rs[s] = jnp.where(hm, rs[s + 1], rs[s])
            idxs[s] = jnp.where(hm, idxs[s + 1], idxs[s])
        rs[K - 1] = jnp.where(hm, NEG, rs[K - 1])

    idxv[...] = idxvec
    pltpu.async_copy(w_hbm.at[idxv], wg, semw).wait()
    ew = jnp.exp(wg[...])
    lanemask = iota < K
    num = _ball(jnp.where(lanemask, valvec * ew, 0.0), jnp.add, iota)
    vsum = _ball(jnp.where(lanemask, valvec, 0.0), jnp.add, iota)
    return num * zinv + vsum / float(K)


@functools.partial(
    pl.kernel,
    mesh=plsc.VectorSubcoreMesh(core_axis_name="c", subcore_axis_name="s"),
    out_type=jax.ShapeDtypeStruct((_NW, L), jnp.float32),
    scratch_types=[
        pltpu.VMEM((2 * N,), jnp.float32),
        pltpu.VMEM((2 * POW,), jnp.int32),
        pltpu.VMEM((L,), jnp.float32),
        pltpu.VMEM((K * L,), jnp.float32),
        pltpu.VMEM((K * L,), jnp.int32),
        pltpu.VMEM((L,), jnp.int32),
        pltpu.VMEM((L,), jnp.float32),
        pltpu.SemaphoreType.DMA,
        pltpu.SemaphoreType.DMA,
        pltpu.SemaphoreType.DMA,
    ],
)
def _sc_kernel(
    x_hbm, m_hbm, w_hbm, out_hbm,
    xv, mv, resv, rsv, isv, idxv, wg, semx, semm, semw,
):
    wid = lax.axis_index("s") * _NC + lax.axis_index("c")
    iota = lax.iota(jnp.int32, L)
    shiftv = 8 * (iota & 3)
    perms = [4 * j + (iota >> 2) for j in range(4)]
    row0 = wid * _RPW

    cx = pltpu.async_copy(x_hbm.at[row0], xv.at[pl.ds(0, N)], semx)
    cm = pltpu.async_copy(m_hbm.at[row0], mv.at[pl.ds(0, POW)], semm)
    pltpu.sync_copy(w_hbm, xv.at[pl.ds(N, N)])

    def zstep(i, accs):
        b = N + i * 64
        return (
            accs[0] + jnp.exp(xv[pl.ds(b, L)]),
            accs[1] + jnp.exp(xv[pl.ds(b + L, L)]),
            accs[2] + jnp.exp(xv[pl.ds(b + 2 * L, L)]),
            accs[3] + jnp.exp(xv[pl.ds(b + 3 * L, L)]),
        )

    z0 = tuple(jnp.zeros((L,), jnp.float32) for _ in range(4))
    z4 = lax.fori_loop(0, N // 64, zstep, z0)
    zinv = 1.0 / _ball(z4[0] + z4[1] + z4[2] + z4[3], jnp.add, iota)

    resvec = jnp.zeros((L,), jnp.float32)
    for r_local in range(_RPW):
        buf = r_local % 2
        cx.wait()
        cm.wait()
        if r_local + 1 < _RPW:
            nrow = row0 + r_local + 1
            nbuf = 1 - buf
            cx = pltpu.async_copy(
                x_hbm.at[nrow], xv.at[pl.ds(nbuf * N, N)], semx
            )
            cm = pltpu.async_copy(
                m_hbm.at[nrow], mv.at[pl.ds(nbuf * POW, POW)], semm
            )
        outb = _row_topk_combine(
            xv, mv, buf * N, buf * POW, w_hbm, zinv,
            rsv, isv, idxv, wg, semw, iota, perms, shiftv,
        )
        resvec = jnp.where(iota == r_local, outb, resvec)

    resv[...] = resvec
    pltpu.sync_copy(resv, out_hbm.at[wid])


def kernel(x, mask, W):
    mp = lax.bitcast_convert_type(
        mask.astype(jnp.int8).reshape(B, POW, 4), jnp.int32
    )
    res = _sc_kernel(x, mp, W.reshape(-1))
    return res[:, :_RPW].reshape(B, 1)


# speedup vs baseline: 1.0930x; 1.0305x over previous
import functools

import jax
import jax.numpy as jnp
from jax import lax
from jax.experimental import pallas as pl
from jax.experimental.pallas import tpu as pltpu
from jax.experimental.pallas import tpu_sc as plsc

N = 32768
B = 128
K = 8
L = 16
POW = N // 4
NEG = float("-inf")

_info = plsc.get_sparse_core_info()
_NC, _NS = _info.num_cores, _info.num_subcores
_NW = _NC * _NS
_RPW = B // _NW

BLK = 256
NBLK = N // BLK

_GATHER_DNUMS = lax.GatherDimensionNumbers(
    offset_dims=(), collapsed_slice_dims=(0,), start_index_map=(0,)
)


def _shuf(v, idx):
    return lax.gather(
        v,
        idx[:, None],
        dimension_numbers=_GATHER_DNUMS,
        slice_sizes=(1,),
        mode=lax.GatherScatterMode.PROMISE_IN_BOUNDS,
    )


def _ball(v, op, iota):
    for sh in (1, 2, 4, 8):
        v = op(v, _shuf(v, iota ^ sh))
    return v


def _row_topk_combine(
    xv, mv, xo, mo, w_hbm, zinv, rsv, isv, idxv, wg, semw, iota, perms, shiftv
):
    for s in range(K):
        rsv[pl.ds(s * L, L)] = jnp.full((L,), NEG, jnp.float32)
        isv[pl.ds(s * L, L)] = jnp.zeros((L,), jnp.int32)

    def blk_body(blk, carry):
        base = blk * BLK
        xb = xo + base
        mb = mo + blk * (BLK // 4)
        mx = jnp.full((L,), NEG)
        for h in range(BLK // 64):
            mw = mv[pl.ds(mb + h * L, L)]
            for j in range(4):
                v = xv[pl.ds(xb + h * 64 + j * L, L)]
                bits = (_shuf(mw, perms[j]) >> shiftv) & 1
                mx = jnp.maximum(mx, jnp.where(bits > 0, v, NEG))
        thr = rsv[pl.ds((K - 1) * L, L)]
        anyv = _ball(
            jnp.where(mx > thr, 1, 0).astype(jnp.int32), jnp.maximum, iota
        )

        @pl.when(anyv[0] > 0)
        def _():
            rs = [rsv[pl.ds(s * L, L)] for s in range(K)]
            idxs = [isv[pl.ds(s * L, L)] for s in range(K)]
            for h in range(BLK // 64):
                mw = mv[pl.ds(mb + h * L, L)]
                for j in range(4):
                    v = xv[pl.ds(xb + h * 64 + j * L, L)]
                    bits = (_shuf(mw, perms[j]) >> shiftv) & 1
                    cv = jnp.where(bits > 0, v, NEG)
                    ci = base + h * 64 + j * L + iota
                    for s in range(K):
                        take = cv > rs[s]
                        nr = jnp.where(take, cv, rs[s])
                        ni = jnp.where(take, ci, idxs[s])
                        cv, ci = (
                            jnp.where(take, rs[s], cv),
                            jnp.where(take, idxs[s], ci),
                        )
                        rs[s], idxs[s] = nr, ni
            for s in range(K):
                rsv[pl.ds(s * L, L)] = rs[s]
                isv[pl.ds(s * L, L)] = idxs[s]

        return carry

    lax.fori_loop(0, NBLK, blk_body, 0)
    rs = [rsv[pl.ds(s * L, L)] for s in range(K)]
    idxs = [isv[pl.ds(s * L, L)] for s in range(K)]

    idxvec = jnp.zeros((L,), jnp.int32)
    valvec = jnp.zeros((L,), jnp.float32)
    for k in range(K):
        gmax = _ball(rs[0], jnp.maximum, iota)
        hit = rs[0] == gmax
        ml = _ball(jnp.where(hit, iota, L), jnp.minimum, iota)
        hm = iota == ml
        ik = _ball(jnp.where(hm, idxs[0], 0), jnp.maximum, iota)
        valvec = jnp.where(iota == k, gmax, valvec)
        idxvec = jnp.where(iota == k, ik, idxvec)
        for s in range(K - 1):
            rs[s] = jnp.where(hm, rs[s + 1], rs[s])
            idxs[s] = jnp.where(hm, idxs[s + 1], idxs[s])
        rs[K - 1] = jnp.where(hm, NEG, rs[K - 1])

    idxv[...] = idxvec
    pltpu.async_copy(w_hbm.at[idxv], wg, semw).wait()
    ew = jnp.exp(wg[...])
    lanemask = iota < K
    num = _ball(jnp.where(lanemask, valvec * ew, 0.0), jnp.add, iota)
    vsum = _ball(jnp.where(lanemask, valvec, 0.0), jnp.add, iota)
    return num * zinv + vsum / float(K)


@functools.partial(
    pl.kernel,
    mesh=plsc.VectorSubcoreMesh(core_axis_name="c", subcore_axis_name="s"),
    out_type=jax.ShapeDtypeStruct((_NW, L), jnp.float32),
    scratch_types=[
        pltpu.VMEM((2 * N,), jnp.float32),
        pltpu.VMEM((2 * POW,), jnp.int32),
        pltpu.VMEM((L,), jnp.float32),
        pltpu.VMEM((K * L,), jnp.float32),
        pltpu.VMEM((K * L,), jnp.int32),
        pltpu.VMEM((L,), jnp.int32),
        pltpu.VMEM((L,), jnp.float32),
        pltpu.SemaphoreType.DMA,
        pltpu.SemaphoreType.DMA,
        pltpu.SemaphoreType.DMA,
    ],
)
def _sc_kernel(
    x_hbm, m_hbm, w_hbm, out_hbm,
    xv, mv, resv, rsv, isv, idxv, wg, semx, semm, semw,
):
    wid = lax.axis_index("s") * _NC + lax.axis_index("c")
    iota = lax.iota(jnp.int32, L)
    shiftv = 8 * (iota & 3)
    perms = [4 * j + (iota >> 2) for j in range(4)]
    row0 = wid * _RPW

    cx = pltpu.async_copy(x_hbm.at[row0], xv.at[pl.ds(0, N)], semx)
    cm = pltpu.async_copy(m_hbm.at[row0], mv.at[pl.ds(0, POW)], semm)
    pltpu.sync_copy(w_hbm, xv.at[pl.ds(N, N)])

    def zstep(i, accs):
        b = N + i * 64
        return (
            accs[0] + jnp.exp(xv[pl.ds(b, L)]),
            accs[1] + jnp.exp(xv[pl.ds(b + L, L)]),
            accs[2] + jnp.exp(xv[pl.ds(b + 2 * L, L)]),
            accs[3] + jnp.exp(xv[pl.ds(b + 3 * L, L)]),
        )

    z0 = tuple(jnp.zeros((L,), jnp.float32) for _ in range(4))
    z4 = lax.fori_loop(0, N // 64, zstep, z0)
    zinv = 1.0 / _ball(z4[0] + z4[1] + z4[2] + z4[3], jnp.add, iota)

    resvec = jnp.zeros((L,), jnp.float32)
    for r_local in range(_RPW):
        buf = r_local % 2
        cx.wait()
        cm.wait()
        if r_local + 1 < _RPW:
            nrow = row0 + r_local + 1
            nbuf = 1 - buf
            cx = pltpu.async_copy(
                x_hbm.at[nrow], xv.at[pl.ds(nbuf * N, N)], semx
            )
            cm = pltpu.async_copy(
                m_hbm.at[nrow], mv.at[pl.ds(nbuf * POW, POW)], semm
            )
        outb = _row_topk_combine(
            xv, mv, buf * N, buf * POW, w_hbm, zinv,
            rsv, isv, idxv, wg, semw, iota, perms, shiftv,
        )
        resvec = jnp.where(iota == r_local, outb, resvec)

    resv[...] = resvec
    pltpu.sync_copy(resv, out_hbm.at[wid])


def kernel(x, mask, W):
    mp = lax.bitcast_convert_type(
        mask.astype(jnp.int8).reshape(B, POW, 4), jnp.int32
    )
    res = _sc_kernel(x, mp, W.reshape(-1))
    return res[:, :_RPW].reshape(B, 1)


# speedup vs baseline: 1.0960x; 1.0028x over previous
import functools

import jax
import jax.numpy as jnp
from jax import lax
from jax.experimental import pallas as pl
from jax.experimental.pallas import tpu as pltpu
from jax.experimental.pallas import tpu_sc as plsc

N = 32768
B = 128
K = 8
L = 16
POW = N // 4
NEG = float("-inf")

_info = plsc.get_sparse_core_info()
_NC, _NS = _info.num_cores, _info.num_subcores
_NW = _NC * _NS
_RPW = B // _NW

BLK = 256
NBLK = N // BLK

_GATHER_DNUMS = lax.GatherDimensionNumbers(
    offset_dims=(), collapsed_slice_dims=(0,), start_index_map=(0,)
)


def _shuf(v, idx):
    return lax.gather(
        v,
        idx[:, None],
        dimension_numbers=_GATHER_DNUMS,
        slice_sizes=(1,),
        mode=lax.GatherScatterMode.PROMISE_IN_BOUNDS,
    )


def _ball(v, op, iota):
    for sh in (1, 2, 4, 8):
        v = op(v, _shuf(v, iota ^ sh))
    return v


def _row_topk_combine(
    xv, mv, xo, mo, w_hbm, zinv, rsv, isv, idxv, wg, semw, iota, perms, shiftv
):
    for s in range(K):
        rsv[pl.ds(s * L, L)] = jnp.full((L,), NEG, jnp.float32)
        isv[pl.ds(s * L, L)] = jnp.zeros((L,), jnp.int32)

    def blk_body(blk, carry):
        base = blk * BLK
        xb = xo + base
        mb = mo + blk * (BLK // 4)
        mxs = [jnp.full((L,), NEG) for _ in range(4)]
        for h in range(BLK // 64):
            mw = mv[pl.ds(mb + h * L, L)]
            for j in range(4):
                v = xv[pl.ds(xb + h * 64 + j * L, L)]
                bits = (_shuf(mw, perms[j]) >> shiftv) & 1
                mxs[j] = jnp.maximum(mxs[j], jnp.where(bits > 0, v, NEG))
        mx = jnp.maximum(
            jnp.maximum(mxs[0], mxs[1]), jnp.maximum(mxs[2], mxs[3])
        )
        thr = rsv[pl.ds((K - 1) * L, L)]
        anyv = _ball(
            jnp.where(mx > thr, 1, 0).astype(jnp.int32), jnp.maximum, iota
        )

        @pl.when(anyv[0] > 0)
        def _():
            rs = [rsv[pl.ds(s * L, L)] for s in range(K)]
            idxs = [isv[pl.ds(s * L, L)] for s in range(K)]
            for h in range(BLK // 64):
                mw = mv[pl.ds(mb + h * L, L)]
                for j in range(4):
                    v = xv[pl.ds(xb + h * 64 + j * L, L)]
                    bits = (_shuf(mw, perms[j]) >> shiftv) & 1
                    cv = jnp.where(bits > 0, v, NEG)
                    ci = base + h * 64 + j * L + iota
                    for s in range(K):
                        take = cv > rs[s]
                        nr = jnp.where(take, cv, rs[s])
                        ni = jnp.where(take, ci, idxs[s])
                        cv, ci = (
                            jnp.where(take, rs[s], cv),
                            jnp.where(take, idxs[s], ci),
                        )
                        rs[s], idxs[s] = nr, ni
            for s in range(K):
                rsv[pl.ds(s * L, L)] = rs[s]
                isv[pl.ds(s * L, L)] = idxs[s]

        return carry

    lax.fori_loop(0, NBLK, blk_body, 0)
    rs = [rsv[pl.ds(s * L, L)] for s in range(K)]
    idxs = [isv[pl.ds(s * L, L)] for s in range(K)]

    idxvec = jnp.zeros((L,), jnp.int32)
    valvec = jnp.zeros((L,), jnp.float32)
    for k in range(K):
        gmax = _ball(rs[0], jnp.maximum, iota)
        hit = rs[0] == gmax
        ml = _ball(jnp.where(hit, iota, L), jnp.minimum, iota)
        hm = iota == ml
        ik = _ball(jnp.where(hm, idxs[0], 0), jnp.maximum, iota)
        valvec = jnp.where(iota == k, gmax, valvec)
        idxvec = jnp.where(iota == k, ik, idxvec)
        for s in range(K - 1):
            rs[s] = jnp.where(hm, rs[s + 1], rs[s])
            idxs[s] = jnp.where(hm, idxs[s + 1], idxs[s])
        rs[K - 1] = jnp.where(hm, NEG, rs[K - 1])

    idxv[...] = idxvec
    pltpu.async_copy(w_hbm.at[idxv], wg, semw).wait()
    ew = jnp.exp(wg[...])
    lanemask = iota < K
    num = _ball(jnp.where(lanemask, valvec * ew, 0.0), jnp.add, iota)
    vsum = _ball(jnp.where(lanemask, valvec, 0.0), jnp.add, iota)
    return num * zinv + vsum / float(K)


@functools.partial(
    pl.kernel,
    mesh=plsc.VectorSubcoreMesh(core_axis_name="c", subcore_axis_name="s"),
    out_type=jax.ShapeDtypeStruct((_NW, L), jnp.float32),
    scratch_types=[
        pltpu.VMEM((2 * N,), jnp.float32),
        pltpu.VMEM((2 * POW,), jnp.int32),
        pltpu.VMEM((L,), jnp.float32),
        pltpu.VMEM((K * L,), jnp.float32),
        pltpu.VMEM((K * L,), jnp.int32),
        pltpu.VMEM((L,), jnp.int32),
        pltpu.VMEM((L,), jnp.float32),
        pltpu.SemaphoreType.DMA,
        pltpu.SemaphoreType.DMA,
        pltpu.SemaphoreType.DMA,
    ],
)
def _sc_kernel(
    x_hbm, m_hbm, w_hbm, out_hbm,
    xv, mv, resv, rsv, isv, idxv, wg, semx, semm, semw,
):
    wid = lax.axis_index("s") * _NC + lax.axis_index("c")
    iota = lax.iota(jnp.int32, L)
    shiftv = 8 * (iota & 3)
    perms = [4 * j + (iota >> 2) for j in range(4)]
    row0 = wid * _RPW

    cx = pltpu.async_copy(x_hbm.at[row0], xv.at[pl.ds(0, N)], semx)
    cm = pltpu.async_copy(m_hbm.at[row0], mv.at[pl.ds(0, POW)], semm)
    pltpu.sync_copy(w_hbm, xv.at[pl.ds(N, N)])

    def zstep(i, accs):
        b = N + i * 64
        return (
            accs[0] + jnp.exp(xv[pl.ds(b, L)]),
            accs[1] + jnp.exp(xv[pl.ds(b + L, L)]),
            accs[2] + jnp.exp(xv[pl.ds(b + 2 * L, L)]),
            accs[3] + jnp.exp(xv[pl.ds(b + 3 * L, L)]),
        )

    z0 = tuple(jnp.zeros((L,), jnp.float32) for _ in range(4))
    z4 = lax.fori_loop(0, N // 64, zstep, z0)
    zinv = 1.0 / _ball(z4[0] + z4[1] + z4[2] + z4[3], jnp.add, iota)

    resvec = jnp.zeros((L,), jnp.float32)
    for r_local in range(_RPW):
        buf = r_local % 2
        cx.wait()
        cm.wait()
        if r_local + 1 < _RPW:
            nrow = row0 + r_local + 1
            nbuf = 1 - buf
            cx = pltpu.async_copy(
                x_hbm.at[nrow], xv.at[pl.ds(nbuf * N, N)], semx
            )
            cm = pltpu.async_copy(
                m_hbm.at[nrow], mv.at[pl.ds(nbuf * POW, POW)], semm
            )
        outb = _row_topk_combine(
            xv, mv, buf * N, buf * POW, w_hbm, zinv,
            rsv, isv, idxv, wg, semw, iota, perms, shiftv,
        )
        resvec = jnp.where(iota == r_local, outb, resvec)

    resv[...] = resvec
    pltpu.sync_copy(resv, out_hbm.at[wid])


def kernel(x, mask, W):
    mp = lax.bitcast_convert_type(
        mask.astype(jnp.int8).reshape(B, POW, 4), jnp.int32
    )
    res = _sc_kernel(x, mp, W.reshape(-1))
    return res[:, :_RPW].reshape(B, 1)
